# Initial kernel scaffold; baseline (speedup 1.0000x reference)
import jax
import jax.numpy as jnp
from jax.experimental import pallas as pl


def kernel(x, table):
    raise NotImplementedError("write your pallas kernel here")


# speedup vs baseline: 1.7195x; 1.7195x over previous
import jax
import jax.numpy as jnp
from jax.experimental import pallas as pl


def _add_kernel(x_ref, t_ref, o_ref):
    o_ref[...] = x_ref[...] + t_ref[...][None, :, :]


def kernel(x, table):
    B, S, E = x.shape
    SB = 256
    grid = (S // SB,)
    return pl.pallas_call(
        _add_kernel,
        grid=grid,
        in_specs=[
            pl.BlockSpec((B, SB, E), lambda j: (0, j, 0)),
            pl.BlockSpec((SB, E), lambda j: (j, 0)),
        ],
        out_specs=pl.BlockSpec((B, SB, E), lambda j: (0, j, 0)),
        out_shape=jax.ShapeDtypeStruct((B, S, E), x.dtype),
    )(x, table[:S])


# speedup vs baseline: 1.7236x; 1.0024x over previous
import jax
import jax.numpy as jnp
from jax.experimental import pallas as pl


def _add_kernel(x_ref, t_ref, o_ref):
    o_ref[...] = x_ref[...] + t_ref[...][None, :, :]


def kernel(x, table):
    B, S, E = x.shape
    SB = 512
    grid = (S // SB,)
    return pl.pallas_call(
        _add_kernel,
        grid=grid,
        in_specs=[
            pl.BlockSpec((B, SB, E), lambda j: (0, j, 0)),
            pl.BlockSpec((SB, E), lambda j: (j, 0)),
        ],
        out_specs=pl.BlockSpec((B, SB, E), lambda j: (0, j, 0)),
        out_shape=jax.ShapeDtypeStruct((B, S, E), x.dtype),
    )(x, table[:S])
